# baseline (speedup 1.0000x reference)
import jax
import jax.numpy as jnp
from jax.experimental import pallas as pl


def _tc_body(idx_ref, tbl_ref, out_ref):
    br = out_ref.shape[0]
    l = out_ref.shape[1]
    v = tbl_ref.shape[0]
    idx = idx_ref[...]
    ids = jax.lax.broadcasted_iota(jnp.int32, (v, br, l), 0)
    oh = (ids == idx[None, :, :]).astype(jnp.bfloat16)
    hi = tbl_ref[...].astype(jnp.bfloat16)
    out_ref[...] = jax.lax.dot_general(
        oh, hi, (((0,), (0,)), ((), ())),
        preferred_element_type=jnp.float32)


def kernel(channel_ids, embedding_table):
    b, l = channel_ids.shape
    v, d = embedding_table.shape
    br = 16
    return pl.pallas_call(
        _tc_body,
        grid=(b // br,),
        in_specs=[
            pl.BlockSpec((br, l), lambda i: (i, 0)),
            pl.BlockSpec((v, d), lambda i: (0, 0)),
        ],
        out_specs=pl.BlockSpec((br, l, d), lambda i: (i, 0, 0)),
        out_shape=jax.ShapeDtypeStruct((b, l, d), jnp.float32),
    )(channel_ids, embedding_table)


# speedup vs baseline: 2.7833x; 2.7833x over previous
import functools

import jax
import jax.numpy as jnp
from jax import lax
from jax.experimental import pallas as pl
from jax.experimental.pallas import tpu as pltpu
from jax.experimental.pallas import tpu_sc as plsc

NC = 2
NS = 16
NW = NC * NS

NBUF = 4
LOOK = 2


def _gather_kernel(n_total, v_rows, d, chunk):
    per_w = n_total // NW
    n_chunks = per_w // chunk
    assert n_chunks % NBUF == 0 and n_chunks >= 3 * NBUF
    mesh = plsc.VectorSubcoreMesh(core_axis_name="c", subcore_axis_name="s")

    @functools.partial(
        pl.kernel,
        mesh=mesh,
        out_type=jax.ShapeDtypeStruct((n_total, d), jnp.float32),
        scratch_types=[
            pltpu.VMEM((per_w,), jnp.int32),
            pltpu.VMEM((NBUF, chunk, d), jnp.float32),
            pltpu.VMEM_SHARED((v_rows, d), jnp.float32),
            pltpu.VMEM_SHARED((NS, 2, chunk, d), jnp.float32),
            [pltpu.SemaphoreType.DMA] * NBUF,
            [pltpu.SemaphoreType.DMA] * NBUF,
        ],
    )
    def k(idx_hbm, table_hbm, out_hbm, idx_v, rows_v, spm_table, spm_stage,
          gsem, ssem):
        s = lax.axis_index("s")
        wid = s * NC + lax.axis_index("c")
        base = wid * per_w
        @pl.when(s == 0)
        def _stage():
            pltpu.sync_copy(table_hbm, spm_table)

        pltpu.sync_copy(idx_hbm.at[pl.ds(base, per_w)], idx_v)
        plsc.subcore_barrier()

        def g_desc(j, b):
            return pltpu.make_async_copy(
                spm_table.at[idx_v.at[pl.ds(j * chunk, chunk)]],
                rows_v.at[b], gsem[b])

        def out_sl(j):
            return out_hbm.at[pl.ds(base + j * chunk, chunk)]

        def s_desc(j, b):
            if b % 2 == 0:
                return pltpu.make_async_copy(rows_v.at[b], out_sl(j), ssem[b])
            return pltpu.make_async_copy(
                spm_stage.at[s, b // 2], out_sl(j), ssem[b])

        def s_start(j, b):
            if b % 2 == 1:
                pltpu.sync_copy(rows_v.at[b], spm_stage.at[s, b // 2])
            s_desc(j, b).start()

        for j in range(LOOK):
            g_desc(j, j % NBUF).start()

        def step(j, u):
            g_desc(j, u).wait()
            s_start(j, u)

        def advance(j, u):
            fb = (u + LOOK) % NBUF
            s_desc(j + LOOK - NBUF, fb).wait()
            g_desc(j + LOOK, fb).start()

        for u in range(NBUF):
            step(u, u)
            if u + LOOK < NBUF:
                g_desc(u + LOOK, u + LOOK).start()
            else:
                advance(u, u)

        def body(r, carry):
            j0 = NBUF * r
            for u in range(NBUF):
                step(j0 + u, u)
                advance(j0 + u, u)
            return carry

        lax.fori_loop(1, (n_chunks // NBUF) - 1, body, 0)

        j0 = n_chunks - NBUF
        for u in range(NBUF):
            step(j0 + u, u)
            if j0 + u + LOOK < n_chunks:
                advance(j0 + u, u)

        for u in range(NBUF):
            s_desc(j0 + u, u).wait()

    return k


def kernel(channel_ids, embedding_table):
    b, l = channel_ids.shape
    v, d = embedding_table.shape
    n_total = b * l
    idx_flat = channel_ids.reshape(n_total)
    out = _gather_kernel(n_total, v, d, 64)(idx_flat, embedding_table)
    return out.reshape(b, l, d)


# speedup vs baseline: 3.2084x; 1.1527x over previous
import functools

import jax
import jax.numpy as jnp
from jax import lax
from jax.experimental import pallas as pl
from jax.experimental.pallas import tpu as pltpu
from jax.experimental.pallas import tpu_sc as plsc

NC = 2
NS = 16
NW = NC * NS

NBUF = 5
LOOK = 3


def _gather_kernel(n_total, v_rows, d, chunk):
    per_w = n_total // NW
    n_chunks = per_w // chunk
    assert n_chunks == 50, "schedule below is specialized to 50 chunks/worker"
    mesh = plsc.VectorSubcoreMesh(core_axis_name="c", subcore_axis_name="s")

    @functools.partial(
        pl.kernel,
        mesh=mesh,
        out_type=jax.ShapeDtypeStruct((n_total, d), jnp.float32),
        scratch_types=[
            pltpu.VMEM((per_w,), jnp.int32),
            pltpu.VMEM((NBUF, chunk, d), jnp.float32),
            pltpu.VMEM_SHARED((v_rows, d), jnp.float32),
            [pltpu.SemaphoreType.DMA] * NBUF,
            [pltpu.SemaphoreType.DMA] * NBUF,
        ],
    )
    def k(idx_hbm, table_hbm, out_hbm, idx_v, rows_v, spm_table, gsem, ssem):
        wid = lax.axis_index("s") * NC + lax.axis_index("c")
        base = wid * per_w
        @pl.when(lax.axis_index("s") == 0)
        def _stage():
            pltpu.sync_copy(table_hbm, spm_table)

        pltpu.sync_copy(idx_hbm.at[pl.ds(base, per_w)], idx_v)
        plsc.subcore_barrier()

        def g_desc(j, b):
            return pltpu.make_async_copy(
                spm_table.at[idx_v.at[pl.ds(j * chunk, chunk)]],
                rows_v.at[b], gsem[b])

        def s_desc(j, b):
            return pltpu.make_async_copy(
                rows_v.at[b], out_hbm.at[pl.ds(base + j * chunk, chunk)],
                ssem[b])

        for j in range(LOOK):
            g_desc(j, j % NBUF).start()

        def step(j, u):
            g_desc(j, u).wait()
            s_desc(j, u).start()

        def advance(j, u):
            fb = (u + LOOK) % NBUF
            s_desc(j + LOOK - NBUF, fb).wait()
            g_desc(j + LOOK, fb).start()

        for u in range(NBUF):
            step(u, u)
            if u + LOOK < NBUF:
                g_desc(u + LOOK, u + LOOK).start()
            else:
                advance(u, u)

        def body(r, carry):
            j0 = NBUF * r
            for u in range(NBUF):
                step(j0 + u, u)
                advance(j0 + u, u)
            return carry

        lax.fori_loop(1, (n_chunks // NBUF) - 1, body, 0)

        j0 = n_chunks - NBUF
        for u in range(NBUF):
            step(j0 + u, u)
            if j0 + u + LOOK < n_chunks:
                advance(j0 + u, u)

        for u in range(NBUF):
            s_desc(j0 + u, u).wait()

    return k


def kernel(channel_ids, embedding_table):
    b, l = channel_ids.shape
    v, d = embedding_table.shape
    n_total = b * l
    idx_flat = channel_ids.reshape(n_total)
    out = _gather_kernel(n_total, v, d, 128)(idx_flat, embedding_table)
    return out.reshape(b, l, d)
